# baseline (speedup 1.0000x reference)
import jax
import jax.numpy as jnp
from jax.experimental import pallas as pl
from jax.experimental.pallas import tpu as pltpu

NUM_EXPERTS = 4
CH = 96
HW = 224
BH = 16
WPAD = 232
HPAD = 240


def _conv_kernel(gate_ref, w_ref, b_ref, xa_ref, xb_ref, out_ref):
    g = gate_ref[0]

    x = jnp.concatenate([xa_ref[0], xb_ref[0]], axis=0)

    accs = []
    for dw in range(3):
        acc = jnp.zeros((BH * WPAD, CH), dtype=jnp.float32)
        for dh in range(3):
            wm = jnp.zeros((CH, CH), dtype=jnp.float32)
            for e in range(NUM_EXPERTS):
                ge = g[0:1, e:e + 1]
                wm = wm + ge * w_ref[e, dh, dw]
            xs = x[dh:dh + BH].reshape(BH * WPAD, CH)
            acc = acc + jnp.dot(xs, wm, preferred_element_type=jnp.float32)
        accs.append(acc.reshape(BH, WPAD, CH))

    bm = jnp.dot(g, b_ref[...], preferred_element_type=jnp.float32)
    out = (accs[0][:, 0:HW] + accs[1][:, 1:HW + 1] + accs[2][:, 2:HW + 2]
           + bm[None, :, :])
    out = jnp.maximum(out, 0.0).reshape(BH * HW, CH)
    eye = (jax.lax.broadcasted_iota(jnp.int32, (CH, CH), 0)
           == jax.lax.broadcasted_iota(jnp.int32, (CH, CH), 1)).astype(jnp.float32)
    out_t = jax.lax.dot_general(
        eye, out, dimension_numbers=(((1,), (1,)), ((), ())),
        preferred_element_type=jnp.float32)
    out_ref[0] = out_t


def kernel(x, gate_values, W, b):
    B = x.shape[0]
    xt = jnp.transpose(x, (0, 2, 3, 1))
    xp = jnp.pad(xt, ((0, 0), (1, HPAD - HW - 1), (1, WPAD - HW - 1), (0, 0)))
    wt = jnp.transpose(W, (0, 3, 4, 2, 1))
    gv = gate_values.reshape(B, 1, NUM_EXPERTS)

    n_h = HW // BH
    out = pl.pallas_call(
        _conv_kernel,
        grid=(B, n_h),
        in_specs=[
            pl.BlockSpec((1, 1, NUM_EXPERTS), lambda bb, h: (bb, 0, 0)),
            pl.BlockSpec((NUM_EXPERTS, 3, 3, CH, CH), lambda bb, h: (0, 0, 0, 0, 0)),
            pl.BlockSpec((NUM_EXPERTS, CH), lambda bb, h: (0, 0)),
            pl.BlockSpec((1, BH, WPAD, CH), lambda bb, h: (bb, h, 0, 0)),
            pl.BlockSpec((1, BH, WPAD, CH), lambda bb, h: (bb, h + 1, 0, 0)),
        ],
        out_specs=pl.BlockSpec((1, CH, BH * HW), lambda bb, h: (bb, 0, h)),
        out_shape=jax.ShapeDtypeStruct((B, CH, HW * HW), jnp.float32),
        compiler_params=pltpu.CompilerParams(
            dimension_semantics=("parallel", "arbitrary"),
        ),
    )(gv, wt, b, xp, xp)
    return out.reshape(B, CH, HW, HW)


# speedup vs baseline: 3.0326x; 3.0326x over previous
import jax
import jax.numpy as jnp
from jax.experimental import pallas as pl
from jax.experimental.pallas import tpu as pltpu

NUM_EXPERTS = 4
CH = 96
HW = 224
BH = 16
WPAD = 256
HPAD = 240


def _conv_kernel(gate_ref, w_ref, b_ref, xa_ref, xb_ref, out_ref):
    g = gate_ref[0]

    slab = jnp.concatenate([xa_ref[0], xb_ref[0]], axis=1)
    slab2 = slab.reshape(CH, 2 * BH * WPAD)

    accs = []
    for dw in range(3):
        acc = jnp.zeros((CH, BH * WPAD), dtype=jnp.float32)
        for dh in range(3):
            wm = jnp.zeros((CH, CH), dtype=jnp.float32)
            for e in range(NUM_EXPERTS):
                ge = g[0:1, e:e + 1]
                wm = wm + ge * w_ref[e, dh, dw]
            xs = slab2[:, dh * WPAD: dh * WPAD + BH * WPAD]
            acc = acc + jnp.dot(wm, xs, preferred_element_type=jnp.float32)
        accs.append(acc.reshape(CH, BH, WPAD))

    bm = jnp.zeros((CH, 1), dtype=jnp.float32)
    for e in range(NUM_EXPERTS):
        bm = bm + g[0:1, e:e + 1] * b_ref[:, e:e + 1]

    out = (accs[0][:, :, 0:HW] + accs[1][:, :, 1:HW + 1]
           + accs[2][:, :, 2:HW + 2] + bm[:, :, None])
    out_ref[0] = jnp.maximum(out, 0.0)


def kernel(x, gate_values, W, b):
    B = x.shape[0]
    xp = jnp.pad(x, ((0, 0), (0, 0), (1, HPAD - HW - 1), (1, WPAD - HW - 1)))
    wt = jnp.transpose(W, (0, 3, 4, 1, 2))
    bt = jnp.transpose(b, (1, 0))
    gv = gate_values.reshape(B, 1, NUM_EXPERTS)

    n_h = HW // BH
    out = pl.pallas_call(
        _conv_kernel,
        grid=(B, n_h),
        in_specs=[
            pl.BlockSpec((1, 1, NUM_EXPERTS), lambda bb, h: (bb, 0, 0)),
            pl.BlockSpec((NUM_EXPERTS, 3, 3, CH, CH), lambda bb, h: (0, 0, 0, 0, 0)),
            pl.BlockSpec((CH, NUM_EXPERTS), lambda bb, h: (0, 0)),
            pl.BlockSpec((1, CH, BH, WPAD), lambda bb, h: (bb, 0, h, 0)),
            pl.BlockSpec((1, CH, BH, WPAD), lambda bb, h: (bb, 0, h + 1, 0)),
        ],
        out_specs=pl.BlockSpec((1, CH, BH, HW), lambda bb, h: (bb, 0, h, 0)),
        out_shape=jax.ShapeDtypeStruct((B, CH, HW, HW), jnp.float32),
        compiler_params=pltpu.CompilerParams(
            dimension_semantics=("parallel", "arbitrary"),
        ),
    )(gv, wt, bt, xp, xp)
    return out
